# Initial kernel scaffold; baseline (speedup 1.0000x reference)
import jax
import jax.numpy as jnp
from jax.experimental import pallas as pl


def kernel(emb, center_idx, neighbor_idx, edge_feats, segment_ids, W1, b1, W2, b2):
    raise NotImplementedError("write your pallas kernel here")


# speedup vs baseline: 2.9688x; 2.9688x over previous
import functools

import jax
import jax.numpy as jnp
from jax import lax
from jax.experimental import pallas as pl
from jax.experimental.pallas import tpu as pltpu
from jax.experimental.pallas import tpu_sc as plsc

_NUM_CORES = 2
_NUM_SUBCORES = 16
_NW = _NUM_CORES * _NUM_SUBCORES
_CHUNK = 128


def _sc_gather(emb, nbr_idx, center_idx):
    t_total = nbr_idx.shape[0]
    d = emb.shape[1]
    n_ctr = center_idx.shape[0]
    per_w = t_total // _NW
    n_chunks = per_w // _CHUNK
    idx3 = nbr_idx.reshape(_NW, n_chunks, _CHUNK)

    mesh = plsc.VectorSubcoreMesh(core_axis_name="c", subcore_axis_name="s",
                                  num_cores=_NUM_CORES,
                                  num_subcores=_NUM_SUBCORES)

    @functools.partial(
        pl.kernel,
        mesh=mesh,
        out_type=(
            jax.ShapeDtypeStruct((t_total, d), jnp.float32),
            jax.ShapeDtypeStruct((n_ctr, d), jnp.float32),
        ),
        scratch_types=[
            pltpu.VMEM((n_chunks, _CHUNK), jnp.int32),
            pltpu.VMEM((_CHUNK, d), jnp.float32),
            pltpu.VMEM((_CHUNK, d), jnp.float32),
            pltpu.VMEM((n_ctr,), jnp.int32),
            pltpu.VMEM((n_ctr, d), jnp.float32),
            pltpu.SemaphoreType.DMA,
            pltpu.SemaphoreType.DMA,
            pltpu.SemaphoreType.DMA,
        ],
    )
    def gather_kernel(emb_hbm, idx_hbm, ctr_idx_hbm, out_hbm, ctr_out_hbm,
                      idx_v, rows0, rows1, ctr_idx_v, ctr_rows,
                      sem0, sem1, sem_c):
        wid = lax.axis_index("s") * _NUM_CORES + lax.axis_index("c")
        base = wid * per_w
        pltpu.sync_copy(idx_hbm.at[wid], idx_v)

        bufs = (rows0, rows1)
        sems = (sem0, sem1)
        handles = [None] * n_chunks
        handles[0] = pltpu.async_copy(emb_hbm.at[idx_v.at[0]], bufs[0], sems[0])
        for j in range(1, n_chunks):
            handles[j] = pltpu.async_copy(
                emb_hbm.at[idx_v.at[j]], bufs[j % 2], sems[j % 2])
            handles[j - 1].wait()
            pltpu.sync_copy(bufs[(j - 1) % 2],
                            out_hbm.at[pl.ds(base + (j - 1) * _CHUNK, _CHUNK)])
        last = n_chunks - 1
        handles[last].wait()
        pltpu.sync_copy(bufs[last % 2],
                        out_hbm.at[pl.ds(base + last * _CHUNK, _CHUNK)])

        @pl.when(wid == 0)
        def _():
            pltpu.sync_copy(ctr_idx_hbm, ctr_idx_v)
            pltpu.async_copy(emb_hbm.at[ctr_idx_v], ctr_rows, sem_c).wait()
            pltpu.sync_copy(ctr_rows, ctr_out_hbm)

    return gather_kernel(emb, idx3, center_idx)


def _tc_mlp(gathered, ctr_rows, edge_feats, seg2, w1, b1r, w2r, b2r):
    t_total, d = gathered.shape
    n_ctr = ctr_rows.shape[0]
    d_e = edge_feats.shape[1]
    d_mid = w1.shape[1]
    blk = 2048
    grid = (t_total // blk,)

    def body(g_ref, c_ref, ef_ref, seg_ref, w1_ref, b1_ref, w2_ref, b2_ref,
             out_ref):
        w1_all = w1_ref[...]
        w1a = w1_all[:d, :]
        w1b = w1_all[d:2 * d, :]
        w1c = w1_all[2 * d:, :]
        cmat = lax.dot_general(
            c_ref[...], w1a, (((1,), (0,)), ((), ())),
            precision=lax.Precision.HIGHEST,
            preferred_element_type=jnp.float32)
        pre = lax.dot_general(
            g_ref[...], w1b, (((1,), (0,)), ((), ())),
            precision=lax.Precision.HIGHEST,
            preferred_element_type=jnp.float32)
        pre += lax.dot_general(
            ef_ref[...], w1c, (((1,), (0,)), ((), ())),
            precision=lax.Precision.HIGHEST,
            preferred_element_type=jnp.float32)
        onehot = (seg_ref[...] ==
                  lax.broadcasted_iota(jnp.int32, (blk, n_ctr), 1)
                  ).astype(jnp.float32)
        pre += lax.dot_general(
            onehot, cmat, (((1,), (0,)), ((), ())),
            precision=lax.Precision.HIGHEST,
            preferred_element_type=jnp.float32)
        h = jnp.maximum(pre + b1_ref[...], 0.0)
        out_ref[...] = (jnp.sum(h * w2_ref[...], axis=1, keepdims=True)
                        + b2_ref[...])

    return pl.pallas_call(
        body,
        grid=grid,
        in_specs=[
            pl.BlockSpec((blk, d), lambda i: (i, 0)),
            pl.BlockSpec((n_ctr, d), lambda i: (0, 0)),
            pl.BlockSpec((blk, d_e), lambda i: (i, 0)),
            pl.BlockSpec((blk, 1), lambda i: (i, 0)),
            pl.BlockSpec(w1.shape, lambda i: (0, 0)),
            pl.BlockSpec((1, d_mid), lambda i: (0, 0)),
            pl.BlockSpec((1, d_mid), lambda i: (0, 0)),
            pl.BlockSpec((1, 1), lambda i: (0, 0)),
        ],
        out_specs=pl.BlockSpec((blk, 1), lambda i: (i, 0)),
        out_shape=jax.ShapeDtypeStruct((t_total, 1), jnp.float32),
    )(gathered, ctr_rows, edge_feats, seg2, w1, b1r, w2r, b2r)


def kernel(emb, center_idx, neighbor_idx, edge_feats, segment_ids,
           W1, b1, W2, b2):
    center_idx = center_idx.astype(jnp.int32)
    gathered, ctr_rows = _sc_gather(emb, neighbor_idx, center_idx)
    out = _tc_mlp(
        gathered, ctr_rows, edge_feats,
        segment_ids.reshape(-1, 1),
        W1,
        b1.reshape(1, -1),
        W2.reshape(1, -1),
        b2.reshape(1, 1),
    )
    return out[:, 0]


# speedup vs baseline: 4.3641x; 1.4700x over previous
import functools

import jax
import jax.numpy as jnp
from jax import lax
from jax.experimental import pallas as pl
from jax.experimental.pallas import tpu as pltpu
from jax.experimental.pallas import tpu_sc as plsc

_NUM_CORES = 2
_NUM_SUBCORES = 16
_NW = _NUM_CORES * _NUM_SUBCORES
_CHUNK = 128


def _sc_gather(emb, nbr_idx, center_idx):
    t_total = nbr_idx.shape[0]
    d = emb.shape[1]
    n_ctr = center_idx.shape[0]
    per_w = t_total // _NW
    n_chunks = per_w // _CHUNK
    idx3 = nbr_idx.reshape(_NW, n_chunks, _CHUNK)

    mesh = plsc.VectorSubcoreMesh(core_axis_name="c", subcore_axis_name="s",
                                  num_cores=_NUM_CORES,
                                  num_subcores=_NUM_SUBCORES)

    @functools.partial(
        pl.kernel,
        mesh=mesh,
        out_type=(
            jax.ShapeDtypeStruct((t_total, d), jnp.float32),
            jax.ShapeDtypeStruct((n_ctr, d), jnp.float32),
        ),
        scratch_types=[
            pltpu.VMEM((n_chunks, _CHUNK), jnp.int32),
            pltpu.VMEM((_CHUNK, d), jnp.float32),
            pltpu.VMEM((_CHUNK, d), jnp.float32),
            pltpu.VMEM((n_ctr,), jnp.int32),
            pltpu.VMEM((n_ctr, d), jnp.float32),
            pltpu.SemaphoreType.DMA,
            pltpu.SemaphoreType.DMA,
            pltpu.SemaphoreType.DMA,
        ],
    )
    def gather_kernel(emb_hbm, idx_hbm, ctr_idx_hbm, out_hbm, ctr_out_hbm,
                      idx_v, rows0, rows1, ctr_idx_v, ctr_rows,
                      sem0, sem1, sem_c):
        wid = lax.axis_index("s") * _NUM_CORES + lax.axis_index("c")
        base = wid * per_w
        pltpu.sync_copy(idx_hbm.at[wid], idx_v)

        bufs = (rows0, rows1)
        sems = (sem0, sem1)
        handles = [None] * n_chunks
        handles[0] = pltpu.async_copy(emb_hbm.at[idx_v.at[0]], bufs[0], sems[0])
        for j in range(1, n_chunks):
            handles[j] = pltpu.async_copy(
                emb_hbm.at[idx_v.at[j]], bufs[j % 2], sems[j % 2])
            handles[j - 1].wait()
            pltpu.sync_copy(bufs[(j - 1) % 2],
                            out_hbm.at[pl.ds(base + (j - 1) * _CHUNK, _CHUNK)])
        last = n_chunks - 1
        handles[last].wait()
        pltpu.sync_copy(bufs[last % 2],
                        out_hbm.at[pl.ds(base + last * _CHUNK, _CHUNK)])

        @pl.when(wid == 0)
        def _():
            pltpu.sync_copy(ctr_idx_hbm, ctr_idx_v)
            pltpu.async_copy(emb_hbm.at[ctr_idx_v], ctr_rows, sem_c).wait()
            pltpu.sync_copy(ctr_rows, ctr_out_hbm)

    return gather_kernel(emb, idx3, center_idx)


def _tc_mlp(gathered, ctr_rows, edge_feats, seg_cols, w1, b1r, w2r, b2r):
    t_total, d = gathered.shape
    n_ctr = ctr_rows.shape[0]
    d_e = edge_feats.shape[1]
    d_mid = w1.shape[1]
    blk, nb = seg_cols.shape

    def body(g_ref, c_ref, ef_ref, seg_ref, w1_ref, b1_ref, w2_ref, b2_ref,
             out_ref):
        i = pl.program_id(0)
        w1_all = w1_ref[...]
        w1a = w1_all[:d, :]
        w1b = w1_all[d:2 * d, :]
        w1c = w1_all[2 * d:, :]
        cmat = lax.dot_general(
            c_ref[...], w1a, (((1,), (0,)), ((), ())),
            preferred_element_type=jnp.float32)
        pre = lax.dot_general(
            g_ref[...], w1b, (((1,), (0,)), ((), ())),
            preferred_element_type=jnp.float32)
        pre += lax.dot_general(
            ef_ref[...], w1c, (((1,), (0,)), ((), ())),
            preferred_element_type=jnp.float32)
        col_sel = lax.broadcasted_iota(jnp.int32, (blk, nb), 1) == i
        seg_col = jnp.sum(jnp.where(col_sel, seg_ref[...], 0), axis=1,
                          keepdims=True)
        onehot = (seg_col ==
                  lax.broadcasted_iota(jnp.int32, (blk, n_ctr), 1)
                  ).astype(jnp.float32)
        pre += lax.dot_general(
            onehot, cmat, (((1,), (0,)), ((), ())),
            preferred_element_type=jnp.float32)
        h = jnp.maximum(pre + b1_ref[...], 0.0)
        col = (jnp.sum(h * w2_ref[...], axis=1, keepdims=True)
               + b2_ref[...])
        colb = jnp.broadcast_to(col, (blk, nb))

        @pl.when(i == 0)
        def _():
            out_ref[...] = jnp.where(col_sel, colb, 0.0)

        @pl.when(i > 0)
        def _():
            out_ref[...] = jnp.where(col_sel, colb, out_ref[...])

    return pl.pallas_call(
        body,
        grid=(nb,),
        in_specs=[
            pl.BlockSpec((blk, d), lambda i: (i, 0)),
            pl.BlockSpec((n_ctr, d), lambda i: (0, 0)),
            pl.BlockSpec((blk, d_e), lambda i: (i, 0)),
            pl.BlockSpec((blk, nb), lambda i: (0, 0)),
            pl.BlockSpec(w1.shape, lambda i: (0, 0)),
            pl.BlockSpec((1, d_mid), lambda i: (0, 0)),
            pl.BlockSpec((1, d_mid), lambda i: (0, 0)),
            pl.BlockSpec((1, 1), lambda i: (0, 0)),
        ],
        out_specs=pl.BlockSpec((blk, nb), lambda i: (0, 0)),
        out_shape=jax.ShapeDtypeStruct((blk, nb), jnp.float32),
    )(gathered, ctr_rows, edge_feats, seg_cols, w1, b1r, w2r, b2r)


_BLK = 2048


def kernel(emb, center_idx, neighbor_idx, edge_feats, segment_ids,
           W1, b1, W2, b2):
    center_idx = center_idx.astype(jnp.int32)
    t_total = neighbor_idx.shape[0]
    nb = t_total // _BLK
    gathered, ctr_rows = _sc_gather(emb, neighbor_idx, center_idx)
    out = _tc_mlp(
        gathered, ctr_rows, edge_feats,
        segment_ids.reshape(nb, _BLK).T,
        W1,
        b1.reshape(1, -1),
        W2.reshape(1, -1),
        b2.reshape(1, 1),
    )
    return out.T.reshape(t_total)


# speedup vs baseline: 4.8145x; 1.1032x over previous
import functools

import jax
import jax.numpy as jnp
from jax import lax
from jax.experimental import pallas as pl
from jax.experimental.pallas import tpu as pltpu
from jax.experimental.pallas import tpu_sc as plsc

_NUM_CORES = 2
_NUM_SUBCORES = 16
_NW = _NUM_CORES * _NUM_SUBCORES
_CHUNK = 128


def _sc_gather(emb, nbr_idx, center_idx):
    t_total = nbr_idx.shape[0]
    d = emb.shape[1]
    n_ctr = center_idx.shape[0]
    per_w = t_total // _NW
    n_chunks = per_w // _CHUNK
    idx3 = nbr_idx.reshape(_NW, n_chunks, _CHUNK)

    mesh = plsc.VectorSubcoreMesh(core_axis_name="c", subcore_axis_name="s",
                                  num_cores=_NUM_CORES,
                                  num_subcores=_NUM_SUBCORES)

    @functools.partial(
        pl.kernel,
        mesh=mesh,
        out_type=(
            jax.ShapeDtypeStruct((t_total, d), jnp.float32),
            jax.ShapeDtypeStruct((n_ctr, d), jnp.float32),
        ),
        scratch_types=[
            pltpu.VMEM((n_chunks, _CHUNK), jnp.int32),
            pltpu.VMEM((_CHUNK, d), jnp.float32),
            pltpu.VMEM((_CHUNK, d), jnp.float32),
            pltpu.VMEM((n_ctr,), jnp.int32),
            pltpu.VMEM((n_ctr, d), jnp.float32),
            pltpu.SemaphoreType.DMA,
            pltpu.SemaphoreType.DMA,
            pltpu.SemaphoreType.DMA,
        ],
    )
    def gather_kernel(emb_hbm, idx_hbm, ctr_idx_hbm, out_hbm, ctr_out_hbm,
                      idx_v, rows0, rows1, ctr_idx_v, ctr_rows,
                      sem0, sem1, sem_c):
        wid = lax.axis_index("s") * _NUM_CORES + lax.axis_index("c")
        base = wid * per_w
        pltpu.sync_copy(idx_hbm.at[wid], idx_v)

        bufs = (rows0, rows1)
        sems = (sem0, sem1)
        handles = [None] * n_chunks
        handles[0] = pltpu.async_copy(emb_hbm.at[idx_v.at[0]], bufs[0], sems[0])
        for j in range(1, n_chunks):
            handles[j] = pltpu.async_copy(
                emb_hbm.at[idx_v.at[j]], bufs[j % 2], sems[j % 2])
            handles[j - 1].wait()
            pltpu.sync_copy(bufs[(j - 1) % 2],
                            out_hbm.at[pl.ds(base + (j - 1) * _CHUNK, _CHUNK)])
        last = n_chunks - 1
        handles[last].wait()
        pltpu.sync_copy(bufs[last % 2],
                        out_hbm.at[pl.ds(base + last * _CHUNK, _CHUNK)])

        @pl.when(wid == 0)
        def _():
            pltpu.sync_copy(ctr_idx_hbm, ctr_idx_v)
            pltpu.async_copy(emb_hbm.at[ctr_idx_v], ctr_rows, sem_c).wait()
            pltpu.sync_copy(ctr_rows, ctr_out_hbm)

    return gather_kernel(emb, idx3, center_idx)


def _dg(a, b, dims):
    return lax.dot_general(a, b, (dims, ((), ())),
                           preferred_element_type=jnp.float32)


def _tc_mlp(gathered, ctr_rows, edge_feats, seg_cols,
            w1a2a, w1a2b, w1b2a, w1b2b, w1c2a, w1c2b, b1d, w2two, b2r):
    t_total, d = gathered.shape
    n_ctr = ctr_rows.shape[0]
    d_e = edge_feats.shape[1]
    blk, nb = seg_cols.shape
    nsteps = nb // 2

    def body(g_ref, c_ref, ef_ref, seg_ref,
             w1a2a_ref, w1a2b_ref, w1b2a_ref, w1b2b_ref,
             w1c2a_ref, w1c2b_ref, b1d_ref, w2two_ref, b2_ref, out_ref):
        i = pl.program_id(0)
        sel16 = (lax.broadcasted_iota(jnp.int32, (nb, 2), 0) ==
                 2 * i + lax.broadcasted_iota(jnp.int32, (nb, 2), 1)
                 ).astype(jnp.float32)
        segpair = _dg(seg_ref[...], sel16, (((1,), (0,))))
        iotaf = lax.broadcasted_iota(jnp.int32, (blk, n_ctr), 1
                                     ).astype(jnp.float32)
        oh_a = (segpair[:, 0:1] == iotaf).astype(jnp.float32)
        oh_b = (segpair[:, 1:2] == iotaf).astype(jnp.float32)

        ctr = c_ref[...]
        cmat_a = _dg(ctr, w1a2a_ref[...], (((1,), (0,))))
        cmat_b = _dg(ctr, w1a2b_ref[...], (((1,), (0,))))

        g = g_ref[...]
        ef = ef_ref[...]
        pre = _dg(g[:blk], w1b2a_ref[...], (((1,), (0,))))
        pre += _dg(g[blk:], w1b2b_ref[...], (((1,), (0,))))
        pre += _dg(ef[:blk], w1c2a_ref[...], (((1,), (0,))))
        pre += _dg(ef[blk:], w1c2b_ref[...], (((1,), (0,))))
        pre += _dg(oh_a, cmat_a, (((1,), (0,))))
        pre += _dg(oh_b, cmat_b, (((1,), (0,))))
        h = jnp.maximum(pre + b1d_ref[...], 0.0)
        pair = _dg(h, w2two_ref[...], (((1,), (0,)))) + b2_ref[...]
        scat16 = (lax.broadcasted_iota(jnp.int32, (2, nb), 1) ==
                  2 * i + lax.broadcasted_iota(jnp.int32, (2, nb), 0)
                  ).astype(jnp.float32)
        scat = _dg(pair, scat16, (((1,), (0,))))

        @pl.when(i == 0)
        def _():
            out_ref[...] = scat

        @pl.when(i > 0)
        def _():
            out_ref[...] += scat

    return pl.pallas_call(
        body,
        grid=(nsteps,),
        in_specs=[
            pl.BlockSpec((2 * blk, d), lambda i: (i, 0)),
            pl.BlockSpec((n_ctr, d), lambda i: (0, 0)),
            pl.BlockSpec((2 * blk, d_e), lambda i: (i, 0)),
            pl.BlockSpec((blk, nb), lambda i: (0, 0)),
            pl.BlockSpec((d, 2 * 64), lambda i: (0, 0)),
            pl.BlockSpec((d, 2 * 64), lambda i: (0, 0)),
            pl.BlockSpec((d, 2 * 64), lambda i: (0, 0)),
            pl.BlockSpec((d, 2 * 64), lambda i: (0, 0)),
            pl.BlockSpec((d_e, 2 * 64), lambda i: (0, 0)),
            pl.BlockSpec((d_e, 2 * 64), lambda i: (0, 0)),
            pl.BlockSpec((1, 2 * 64), lambda i: (0, 0)),
            pl.BlockSpec((2 * 64, 2), lambda i: (0, 0)),
            pl.BlockSpec((1, 1), lambda i: (0, 0)),
        ],
        out_specs=pl.BlockSpec((blk, nb), lambda i: (0, 0)),
        out_shape=jax.ShapeDtypeStruct((blk, nb), jnp.float32),
    )(gathered, ctr_rows, edge_feats, seg_cols,
      w1a2a, w1a2b, w1b2a, w1b2b, w1c2a, w1c2b, b1d, w2two, b2r)


_BLK = 2048


def kernel(emb, center_idx, neighbor_idx, edge_feats, segment_ids,
           W1, b1, W2, b2):
    center_idx = center_idx.astype(jnp.int32)
    t_total = neighbor_idx.shape[0]
    d = emb.shape[1]
    d_mid = W1.shape[1]
    nb = t_total // _BLK
    gathered, ctr_rows = _sc_gather(emb, neighbor_idx, center_idx)

    w1a, w1b, w1c = W1[:d], W1[d:2 * d], W1[2 * d:]
    z = jnp.zeros_like
    pad_r = lambda w: jnp.concatenate([w, z(w)], axis=1)
    pad_l = lambda w: jnp.concatenate([z(w), w], axis=1)
    w2two = jnp.zeros((2 * d_mid, 2), jnp.float32)
    w2two = w2two.at[:d_mid, 0].set(W2[:, 0]).at[d_mid:, 1].set(W2[:, 0])
    b1d = jnp.concatenate([b1, b1]).reshape(1, 2 * d_mid)

    out = _tc_mlp(
        gathered, ctr_rows, edge_feats,
        segment_ids.reshape(nb, _BLK).T.astype(jnp.float32),
        pad_r(w1a), pad_l(w1a), pad_r(w1b), pad_l(w1b),
        pad_r(w1c), pad_l(w1c), b1d, w2two,
        b2.reshape(1, 1),
    )
    return out.T.reshape(t_total)
